# Initial kernel scaffold; baseline (speedup 1.0000x reference)
import functools

import jax
import jax.numpy as jnp
from jax import lax
from jax.experimental import pallas as pl
from jax.experimental.pallas import tpu as pltpu
from jax.experimental.pallas import tpu_sc as plsc

N = 10000
D = 128
NEG_SLOPE = 0.2

_info = plsc.get_sparse_core_info()
NC = _info.num_cores
NS = _info.num_subcores
NW = NC * NS
CH = 128

ROWS_PER_SUB = N // NS
DEN_PAD = ((N + 8 * NS - 1) // (8 * NS)) * (8 * NS)
DEN_PER_SUB = DEN_PAD // NS


def _proj_kernel(x_ref, w_ref, as_ref, ad_ref, h_ref, s_ref, d_ref):
    h = jnp.dot(x_ref[...], w_ref[...], preferred_element_type=jnp.float32)
    h_ref[...] = h
    s_ref[...] = jnp.dot(h, as_ref[...], preferred_element_type=jnp.float32)
    d_ref[...] = jnp.dot(h, ad_ref[...], preferred_element_type=jnp.float32)


def _project(x, W, a_src, a_dst):
    B = 2000
    grid = (N // B,)
    return pl.pallas_call(
        _proj_kernel,
        grid=grid,
        in_specs=[
            pl.BlockSpec((B, D), lambda i: (i, 0)),
            pl.BlockSpec((D, D), lambda i: (0, 0)),
            pl.BlockSpec((D, 1), lambda i: (0, 0)),
            pl.BlockSpec((D, 1), lambda i: (0, 0)),
        ],
        out_specs=[
            pl.BlockSpec((B, D), lambda i: (i, 0)),
            pl.BlockSpec((B, 1), lambda i: (i, 0)),
            pl.BlockSpec((B, 1), lambda i: (i, 0)),
        ],
        out_shape=[
            jax.ShapeDtypeStruct((N, D), jnp.float32),
            jax.ShapeDtypeStruct((N, 1), jnp.float32),
            jax.ShapeDtypeStruct((N, 1), jnp.float32),
        ],
    )(x, W, a_src.reshape(D, 1), a_dst.reshape(D, 1))


def _fin_kernel(p_ref, d0_ref, d1_ref, as_ref, ad_ref, h_ref, b_ref, o_ref):
    a = as_ref[...] + ad_ref[...]
    e = jnp.where(a >= 0, a, NEG_SLOPE * a)
    ws = jnp.exp(e)
    p = p_ref[...]
    num = p[0] + p[1] + ws * h_ref[...]
    den = d0_ref[...] + d1_ref[...] + ws
    o_ref[...] = num / den + b_ref[...]


def _finalize(parts, d0, d1, als, ald, h, bias):
    B = 2000
    grid = (N // B,)
    return pl.pallas_call(
        _fin_kernel,
        grid=grid,
        in_specs=[
            pl.BlockSpec((2, B, D), lambda i: (0, i, 0)),
            pl.BlockSpec((B, 1), lambda i: (i, 0)),
            pl.BlockSpec((B, 1), lambda i: (i, 0)),
            pl.BlockSpec((B, 1), lambda i: (i, 0)),
            pl.BlockSpec((B, 1), lambda i: (i, 0)),
            pl.BlockSpec((B, D), lambda i: (i, 0)),
            pl.BlockSpec((1, D), lambda i: (0, 0)),
        ],
        out_specs=pl.BlockSpec((B, D), lambda i: (i, 0)),
        out_shape=jax.ShapeDtypeStruct((N, D), jnp.float32),
    )(parts, d0, d1, als, ald, h, bias.reshape(1, D))


def _make_sc_kernel(E, EPAD):
    PT = EPAD // NW
    NCHUNK = PT // CH
    mesh = plsc.VectorSubcoreMesh(core_axis_name="c", subcore_axis_name="s")

    @functools.partial(
        pl.kernel,
        mesh=mesh,
        out_type=[
            jax.ShapeDtypeStruct((NC, N, D), jnp.float32),
            jax.ShapeDtypeStruct((NC, DEN_PAD), jnp.float32),
        ],
        scratch_types=[
            pltpu.VMEM((N,), jnp.float32),
            pltpu.VMEM((N,), jnp.float32),
            pltpu.VMEM((CH,), jnp.int32),
            pltpu.VMEM((CH,), jnp.int32),
            pltpu.VMEM((CH, D), jnp.float32),
            pltpu.VMEM((CH,), jnp.float32),
            pltpu.VMEM_SHARED((N, D), jnp.float32),
            pltpu.VMEM_SHARED((DEN_PAD,), jnp.float32),
            pltpu.SemaphoreType.DMA,
        ],
    )
    def sc_edges(as_hbm, ad_hbm, src_hbm, dst_hbm, h_hbm, z2_hbm, z1_hbm,
                 outp_hbm, denp_hbm,
                 as_v, ad_v, sidx, didx, rows, wbuf, acc_sh, den_sh, sem):
        c = lax.axis_index("c")
        s = lax.axis_index("s")
        wid = s * NC + c

        pltpu.sync_copy(as_hbm, as_v)
        pltpu.sync_copy(ad_hbm, ad_v)

        pltpu.sync_copy(z2_hbm.at[pl.ds(s * ROWS_PER_SUB, ROWS_PER_SUB)],
                        acc_sh.at[pl.ds(s * ROWS_PER_SUB, ROWS_PER_SUB)])
        pltpu.sync_copy(z1_hbm.at[pl.ds(s * DEN_PER_SUB, DEN_PER_SUB)],
                        den_sh.at[pl.ds(s * DEN_PER_SUB, DEN_PER_SUB)])
        plsc.subcore_barrier()

        def chunk_body(k, carry):
            base = wid * PT + k * CH
            pltpu.sync_copy(src_hbm.at[pl.ds(base, CH)], sidx)
            pltpu.sync_copy(dst_hbm.at[pl.ds(base, CH)], didx)
            gather = pltpu.async_copy(h_hbm.at[sidx], rows, sem)
            for g in range(CH // 16):
                si = sidx[pl.ds(g * 16, 16)]
                di = didx[pl.ds(g * 16, 16)]
                a = plsc.load_gather(as_v, [si]) + plsc.load_gather(ad_v, [di])
                e = jnp.where(a >= 0, a, NEG_SLOPE * a)
                wv = jnp.exp(e)
                gid = base + g * 16 + lax.iota(jnp.int32, 16)
                wv = jnp.where(gid < E, wv, 0.0)
                wbuf[pl.ds(g * 16, 16)] = wv
            gather.wait()
            for r in range(CH):
                wsp = plsc.load_gather(wbuf, [jnp.full((16,), r, jnp.int32)])
                for cc in range(D // 16):
                    rows[r, pl.ds(cc * 16, 16)] = rows[r, pl.ds(cc * 16, 16)] * wsp
            pltpu.sync_copy(rows, acc_sh.at[didx], add=True)
            pltpu.sync_copy(wbuf, den_sh.at[didx], add=True)
            return carry

        lax.fori_loop(0, NCHUNK, chunk_body, 0)
        plsc.subcore_barrier()

        pltpu.sync_copy(acc_sh.at[pl.ds(s * ROWS_PER_SUB, ROWS_PER_SUB)],
                        outp_hbm.at[c, pl.ds(s * ROWS_PER_SUB, ROWS_PER_SUB)])
        pltpu.sync_copy(den_sh.at[pl.ds(s * DEN_PER_SUB, DEN_PER_SUB)],
                        denp_hbm.at[c, pl.ds(s * DEN_PER_SUB, DEN_PER_SUB)])

    return sc_edges


def kernel(x, edge_index, W, a_src, a_dst, bias):
    E = edge_index.shape[1]
    EPAD = ((E + NW * CH - 1) // (NW * CH)) * (NW * CH)

    h, als, ald = _project(x, W, a_src, a_dst)

    pad = EPAD - E
    srcp = jnp.concatenate([edge_index[0], jnp.zeros((pad,), jnp.int32)])
    dstp = jnp.concatenate([edge_index[1], jnp.zeros((pad,), jnp.int32)])

    z2 = jnp.zeros((N, D), jnp.float32)
    z1 = jnp.zeros((DEN_PAD,), jnp.float32)

    sc_edges = _make_sc_kernel(E, EPAD)
    parts, denp = sc_edges(als.reshape(N), ald.reshape(N), srcp, dstp, h,
                           z2, z1)

    d0 = denp[0, :N].reshape(N, 1)
    d1 = denp[1, :N].reshape(N, 1)
    return _finalize(parts, d0, d1, als, ald, h, bias)


# speedup vs baseline: 19.1865x; 19.1865x over previous
import functools

import jax
import jax.numpy as jnp
from jax import lax
from jax.experimental import pallas as pl
from jax.experimental.pallas import tpu as pltpu
from jax.experimental.pallas import tpu_sc as plsc

N = 10000
D = 128
NEG_SLOPE = 0.2

_info = plsc.get_sparse_core_info()
NC = _info.num_cores
NS = _info.num_subcores
NW = NC * NS
CH = 128

NPAD = ((N + 8 * NS - 1) // (8 * NS)) * (8 * NS)
ROWS_PER_SUB = NPAD // NS
DEN_PAD = NPAD
DEN_PER_SUB = DEN_PAD // NS


def _proj_kernel(x_ref, w_ref, as_ref, ad_ref, h_ref, s_ref, d_ref):
    h = jnp.dot(x_ref[...], w_ref[...], preferred_element_type=jnp.float32)
    h_ref[...] = h
    s_ref[...] = jnp.dot(h, as_ref[...], preferred_element_type=jnp.float32)
    d_ref[...] = jnp.dot(h, ad_ref[...], preferred_element_type=jnp.float32)


def _project(x, W, a_src, a_dst):
    B = 2000
    grid = (N // B,)
    return pl.pallas_call(
        _proj_kernel,
        grid=grid,
        in_specs=[
            pl.BlockSpec((B, D), lambda i: (i, 0)),
            pl.BlockSpec((D, D), lambda i: (0, 0)),
            pl.BlockSpec((D, 1), lambda i: (0, 0)),
            pl.BlockSpec((D, 1), lambda i: (0, 0)),
        ],
        out_specs=[
            pl.BlockSpec((B, D), lambda i: (i, 0)),
            pl.BlockSpec((B, 1), lambda i: (i, 0)),
            pl.BlockSpec((B, 1), lambda i: (i, 0)),
        ],
        out_shape=[
            jax.ShapeDtypeStruct((N, D), jnp.float32),
            jax.ShapeDtypeStruct((N, 1), jnp.float32),
            jax.ShapeDtypeStruct((N, 1), jnp.float32),
        ],
    )(x, W, a_src.reshape(D, 1), a_dst.reshape(D, 1))


def _fin_kernel(p_ref, d0_ref, d1_ref, as_ref, ad_ref, h_ref, b_ref, o_ref):
    a = as_ref[...] + ad_ref[...]
    e = jnp.where(a >= 0, a, NEG_SLOPE * a)
    ws = jnp.exp(e)
    p = p_ref[...]
    num = p[0] + p[1] + ws * h_ref[...]
    den = d0_ref[...] + d1_ref[...] + ws
    o_ref[...] = num / den + b_ref[...]


def _finalize(parts, d0, d1, als, ald, h, bias):
    B = 2000
    grid = (N // B,)
    return pl.pallas_call(
        _fin_kernel,
        grid=grid,
        in_specs=[
            pl.BlockSpec((2, B, D), lambda i: (0, i, 0)),
            pl.BlockSpec((B, 1), lambda i: (i, 0)),
            pl.BlockSpec((B, 1), lambda i: (i, 0)),
            pl.BlockSpec((B, 1), lambda i: (i, 0)),
            pl.BlockSpec((B, 1), lambda i: (i, 0)),
            pl.BlockSpec((B, D), lambda i: (i, 0)),
            pl.BlockSpec((1, D), lambda i: (0, 0)),
        ],
        out_specs=pl.BlockSpec((B, D), lambda i: (i, 0)),
        out_shape=jax.ShapeDtypeStruct((N, D), jnp.float32),
    )(parts, d0, d1, als, ald, h, bias.reshape(1, D))


NBUF = 2


def _make_sc_kernel(E, EPAD):
    PT = EPAD // NW
    NCHUNK = PT // CH
    mesh = plsc.VectorSubcoreMesh(core_axis_name="c", subcore_axis_name="s")

    @functools.partial(
        pl.kernel,
        mesh=mesh,
        compiler_params=pltpu.CompilerParams(needs_layout_passes=False),
        out_type=[
            jax.ShapeDtypeStruct((NC, NPAD, D), jnp.float32),
            jax.ShapeDtypeStruct((NC * DEN_PAD,), jnp.float32),
        ],
        scratch_types=[
            pltpu.VMEM((NBUF, CH), jnp.int32),
            pltpu.VMEM((NBUF, CH), jnp.int32),
            pltpu.VMEM((NBUF, CH), jnp.float32),
            pltpu.VMEM((NBUF, CH), jnp.float32),
            pltpu.VMEM((NBUF, CH, D), jnp.float32),
            pltpu.VMEM((NBUF, CH), jnp.float32),
            pltpu.VMEM((16,), jnp.int32),
            pltpu.VMEM((DEN_PER_SUB,), jnp.float32),
            pltpu.VMEM_SHARED((NPAD, D), jnp.float32),
            pltpu.VMEM_SHARED((DEN_PAD,), jnp.float32),
            pltpu.SemaphoreType.DMA,
            pltpu.SemaphoreType.DMA,
            pltpu.SemaphoreType.DMA,
        ],
    )
    def sc_edges(as_hbm, ad_hbm, src_hbm, dst_hbm, h_hbm, z2_hbm, z1_hbm,
                 outp_hbm, denp_hbm,
                 sidx, didx, asb, adb, rows, wbuf, dumi, den_v, acc_sh,
                 den_sh, sem, sema, semb):
        c = lax.axis_index("c")
        s = lax.axis_index("s")
        wid = s * NC + c

        dumi[...] = jnp.full((16,), NPAD - 16, jnp.int32) + lax.iota(jnp.int32, 16)

        pltpu.sync_copy(z2_hbm.at[pl.ds(s * ROWS_PER_SUB, ROWS_PER_SUB)],
                        acc_sh.at[pl.ds(s * ROWS_PER_SUB, ROWS_PER_SUB)])
        pltpu.sync_copy(z1_hbm.at[pl.ds(s * DEN_PER_SUB, DEN_PER_SUB)], den_v)
        pltpu.sync_copy(den_v, den_sh.at[pl.ds(s * DEN_PER_SUB, DEN_PER_SUB)])
        plsc.subcore_barrier()

        def process(k, b):
            base = wid * PT + k * CH
            pltpu.sync_copy(src_hbm.at[pl.ds(base, CH)], sidx.at[b])
            pltpu.sync_copy(dst_hbm.at[pl.ds(base, CH)], didx.at[b])
            gather = pltpu.async_copy(h_hbm.at[sidx.at[b]], rows.at[b], sem)
            ga = pltpu.async_copy(as_hbm.at[sidx.at[b]], asb.at[b], sema)
            gb = pltpu.async_copy(ad_hbm.at[didx.at[b]], adb.at[b], semb)
            ga.wait()
            gb.wait()
            for g in range(CH // 16):
                a = asb[b, pl.ds(g * 16, 16)] + adb[b, pl.ds(g * 16, 16)]
                e = jnp.where(a >= 0, a, NEG_SLOPE * a)
                wv = jnp.exp(e)
                gid = base + g * 16 + lax.iota(jnp.int32, 16)
                wv = jnp.where(gid < E, wv, 0.0)
                wbuf[b, pl.ds(g * 16, 16)] = wv
            gather.wait()

            def rowgrp(g, carry):
                for i in range(16):
                    r = g * 16 + i
                    wsp = plsc.load_gather(wbuf.at[b],
                                           [jnp.full((16,), r, jnp.int32)])
                    for cc in range(D // 16):
                        rows[b, r, pl.ds(cc * 16, 16)] = (
                            rows[b, r, pl.ds(cc * 16, 16)] * wsp)
                return carry
            lax.fori_loop(0, CH // 16, rowgrp, 0)
            pltpu.sync_copy(rows.at[b], acc_sh.at[didx.at[b]], add=True)
            pltpu.sync_copy(wbuf.at[b], den_sh.at[didx.at[b]], add=True)

        def chunk_body(k, carry):
            for b in range(NBUF):
                @pl.when(lax.rem(k, NBUF) == b)
                def _():
                    process(k, b)
            return carry

        lax.fori_loop(0, NCHUNK, chunk_body, 0)

        pltpu.sync_copy(rows.at[0, pl.ds(0, 16)], acc_sh.at[dumi], add=True)
        pltpu.sync_copy(wbuf.at[0, pl.ds(0, 16)], den_sh.at[dumi], add=True)
        plsc.subcore_barrier()

        pltpu.sync_copy(acc_sh.at[pl.ds(s * ROWS_PER_SUB, ROWS_PER_SUB)],
                        outp_hbm.at[c, pl.ds(s * ROWS_PER_SUB, ROWS_PER_SUB)])
        pltpu.sync_copy(den_sh.at[pl.ds(s * DEN_PER_SUB, DEN_PER_SUB)], den_v)
        pltpu.sync_copy(den_v,
                        denp_hbm.at[pl.ds(c * DEN_PAD + s * DEN_PER_SUB,
                                          DEN_PER_SUB)])

    return sc_edges


def kernel(x, edge_index, W, a_src, a_dst, bias):
    E = edge_index.shape[1]
    EPAD = ((E + NW * CH - 1) // (NW * CH)) * (NW * CH)

    h, als, ald = _project(x, W, a_src, a_dst)

    pad = EPAD - E
    srcp = jnp.concatenate([edge_index[0], jnp.zeros((pad,), jnp.int32)])
    dstp = jnp.concatenate([edge_index[1], jnp.zeros((pad,), jnp.int32)])

    z2 = jnp.zeros((NPAD, D), jnp.float32)
    z1 = jnp.zeros((DEN_PAD,), jnp.float32)

    sc_edges = _make_sc_kernel(E, EPAD)
    parts, denp = sc_edges(als.reshape(N), ald.reshape(N), srcp, dstp, h,
                           z2, z1)

    d0 = denp[:N].reshape(N, 1)
    d1 = denp[DEN_PAD:DEN_PAD + N].reshape(N, 1)
    return _finalize(parts, d0, d1, als, ald, h, bias)


# speedup vs baseline: 45.3982x; 2.3662x over previous
import functools

import jax
import jax.numpy as jnp
from jax import lax
from jax.experimental import pallas as pl
from jax.experimental.pallas import tpu as pltpu
from jax.experimental.pallas import tpu_sc as plsc

N = 10000
D = 128
NEG_SLOPE = 0.2

_info = plsc.get_sparse_core_info()
NC = _info.num_cores
NS = _info.num_subcores
NW = NC * NS
CH = 80

NPAD = ((N + 8 * NS - 1) // (8 * NS)) * (8 * NS)
ROWS_PER_SUB = NPAD // NS
DEN_PAD = NPAD
DEN_PER_SUB = DEN_PAD // NS


def _proj_kernel(x_ref, w_ref, as_ref, ad_ref, h_ref, s_ref, d_ref):
    h = jnp.dot(x_ref[...], w_ref[...], preferred_element_type=jnp.float32)
    h_ref[...] = h
    s_ref[...] = jnp.dot(h, as_ref[...], preferred_element_type=jnp.float32)
    d_ref[...] = jnp.dot(h, ad_ref[...], preferred_element_type=jnp.float32)


def _project(x, W, a_src, a_dst):
    B = 2000
    grid = (N // B,)
    return pl.pallas_call(
        _proj_kernel,
        grid=grid,
        in_specs=[
            pl.BlockSpec((B, D), lambda i: (i, 0)),
            pl.BlockSpec((D, D), lambda i: (0, 0)),
            pl.BlockSpec((D, 1), lambda i: (0, 0)),
            pl.BlockSpec((D, 1), lambda i: (0, 0)),
        ],
        out_specs=[
            pl.BlockSpec((B, D), lambda i: (i, 0)),
            pl.BlockSpec((B, 1), lambda i: (i, 0)),
            pl.BlockSpec((B, 1), lambda i: (i, 0)),
        ],
        out_shape=[
            jax.ShapeDtypeStruct((N, D), jnp.float32),
            jax.ShapeDtypeStruct((N, 1), jnp.float32),
            jax.ShapeDtypeStruct((N, 1), jnp.float32),
        ],
    )(x, W, a_src.reshape(D, 1), a_dst.reshape(D, 1))


def _fin_kernel(p_ref, d0_ref, d1_ref, as_ref, ad_ref, h_ref, b_ref, o_ref):
    a = as_ref[...] + ad_ref[...]
    e = jnp.where(a >= 0, a, NEG_SLOPE * a)
    ws = jnp.exp(e)
    p = p_ref[...]
    num = p[0] + p[1] + ws * h_ref[...]
    den = d0_ref[...] + d1_ref[...] + ws
    o_ref[...] = num / den + b_ref[...]


def _finalize(parts, d0, d1, als, ald, h, bias):
    B = 2000
    grid = (N // B,)
    return pl.pallas_call(
        _fin_kernel,
        grid=grid,
        in_specs=[
            pl.BlockSpec((2, B, D), lambda i: (0, i, 0)),
            pl.BlockSpec((B, 1), lambda i: (i, 0)),
            pl.BlockSpec((B, 1), lambda i: (i, 0)),
            pl.BlockSpec((B, 1), lambda i: (i, 0)),
            pl.BlockSpec((B, 1), lambda i: (i, 0)),
            pl.BlockSpec((B, D), lambda i: (i, 0)),
            pl.BlockSpec((1, D), lambda i: (0, 0)),
        ],
        out_specs=pl.BlockSpec((B, D), lambda i: (i, 0)),
        out_shape=jax.ShapeDtypeStruct((N, D), jnp.float32),
    )(parts, d0, d1, als, ald, h, bias.reshape(1, D))


NBUF = 3
NIB = 6


def _make_sc_kernel(E, NCHUNK):
    PT = NCHUNK * CH
    mesh = plsc.VectorSubcoreMesh(core_axis_name="c", subcore_axis_name="s")

    @functools.partial(
        pl.kernel,
        mesh=mesh,
        compiler_params=pltpu.CompilerParams(needs_layout_passes=False),
        out_type=[
            jax.ShapeDtypeStruct((NC, NPAD, D), jnp.float32),
            jax.ShapeDtypeStruct((NC * DEN_PAD,), jnp.float32),
        ],
        scratch_types=[
            pltpu.VMEM((PT,), jnp.int32),
            pltpu.VMEM((NIB, CH), jnp.int32),
            pltpu.VMEM((NBUF, CH), jnp.float32),
            pltpu.VMEM((NBUF, CH), jnp.float32),
            pltpu.VMEM((NBUF, CH, D), jnp.float32),
            pltpu.VMEM((NBUF, CH), jnp.float32),
            pltpu.VMEM((16,), jnp.int32),
            pltpu.VMEM((DEN_PER_SUB,), jnp.float32),
            pltpu.VMEM_SHARED((NPAD, D), jnp.float32),
            pltpu.VMEM_SHARED((DEN_PAD,), jnp.float32),
            pltpu.SemaphoreType.DMA,
            pltpu.SemaphoreType.DMA,
            pltpu.SemaphoreType.DMA,
            pltpu.SemaphoreType.DMA,
            pltpu.SemaphoreType.DMA,
            pltpu.SemaphoreType.DMA,
            pltpu.SemaphoreType.DMA,
            pltpu.SemaphoreType.DMA,
            pltpu.SemaphoreType.DMA,
            pltpu.SemaphoreType.DMA,
            pltpu.SemaphoreType.DMA,
            pltpu.SemaphoreType.DMA,
            pltpu.SemaphoreType.DMA,
            pltpu.SemaphoreType.DMA,
            pltpu.SemaphoreType.DMA,
            pltpu.SemaphoreType.DMA,
            pltpu.SemaphoreType.DMA,
            pltpu.SemaphoreType.DMA,
            pltpu.SemaphoreType.DMA,
            pltpu.SemaphoreType.DMA,
            pltpu.SemaphoreType.DMA,
        ],
    )
    def sc_edges(as_hbm, ad_hbm, src_hbm, dst_hbm, h_hbm, z2_hbm, z1_hbm,
                 outp_hbm, denp_hbm,
                 sidx, didx, asb, adb, rows, wbuf, dumi, den_v, acc_sh, den_sh,
                 rs0, rs1, rs2, as0, as1, as2, ad0, ad1, ad2,
                 sa0, sa1, sa2, sd0, sd1, sd2,
                 is0, is1, is2, is3, is4, is5):
        rsem = [rs0, rs1, rs2]
        asem = [as0, as1, as2]
        dsem = [ad0, ad1, ad2]
        sasem = [sa0, sa1, sa2]
        sdsem = [sd0, sd1, sd2]
        isem = [is0, is1, is2, is3, is4, is5]

        c = lax.axis_index("c")
        s = lax.axis_index("s")
        wid = s * NC + c
        base0 = wid * PT

        dumi[...] = jnp.full((16,), NPAD - 16, jnp.int32) + lax.iota(jnp.int32, 16)

        pltpu.sync_copy(src_hbm.at[pl.ds(base0, PT)], sidx)
        pltpu.sync_copy(dst_hbm.at[pl.ds(base0, CH)], didx.at[0])
        pltpu.async_copy(dst_hbm.at[pl.ds(base0 + CH, CH)], didx.at[1],
                         isem[1])

        pltpu.sync_copy(z2_hbm.at[pl.ds(s * ROWS_PER_SUB, ROWS_PER_SUB)],
                        acc_sh.at[pl.ds(s * ROWS_PER_SUB, ROWS_PER_SUB)])
        pltpu.sync_copy(z1_hbm.at[pl.ds(s * DEN_PER_SUB, DEN_PER_SUB)], den_v)
        pltpu.sync_copy(den_v, den_sh.at[pl.ds(s * DEN_PER_SUB, DEN_PER_SUB)])
        plsc.subcore_barrier()

        def issue_gathers(k, b, dk):
            src_ix = sidx.at[pl.ds(k * CH, CH)]
            pltpu.async_copy(h_hbm.at[src_ix], rows.at[b], rsem[b])
            pltpu.async_copy(as_hbm.at[src_ix], asb.at[b], asem[b])
            pltpu.async_copy(ad_hbm.at[didx.at[dk]], adb.at[b], dsem[b])

        def wait_gathers(k, b, dk):
            src_ix = sidx.at[pl.ds(k * CH, CH)]
            pltpu.make_async_copy(h_hbm.at[src_ix], rows.at[b],
                                  rsem[b]).wait()
            pltpu.make_async_copy(as_hbm.at[src_ix], asb.at[b],
                                  asem[b]).wait()
            pltpu.make_async_copy(ad_hbm.at[didx.at[dk]], adb.at[b],
                                  dsem[b]).wait()

        def issue_scatters(b, dk):
            pltpu.async_copy(rows.at[b], acc_sh.at[didx.at[dk]], sasem[b],
                             add=True)
            pltpu.async_copy(wbuf.at[b], den_sh.at[didx.at[dk]], sdsem[b],
                             add=True)

        def wait_scatters(b, dk):
            pltpu.make_async_copy(rows.at[b], acc_sh.at[didx.at[dk]],
                                  sasem[b]).wait()
            pltpu.make_async_copy(wbuf.at[b], den_sh.at[didx.at[dk]],
                                  sdsem[b]).wait()

        issue_gathers(0, 0, 0)

        def process(k, u):
            b = u % NBUF
            bn = (b + 1) % NBUF
            dk = u
            dk1 = (u + 1) % NIB
            dk2 = (u + 2) % NIB
            dkm2 = (u + 4) % NIB

            @pl.when(k >= 2)
            def _():
                wait_scatters(bn, dkm2)

            @pl.when(k + 2 < NCHUNK)
            def _():
                pltpu.async_copy(
                    dst_hbm.at[pl.ds(base0 + (k + 2) * CH, CH)],
                    didx.at[dk2], isem[dk2])

            @pl.when(k + 1 < NCHUNK)
            def _():
                pltpu.make_async_copy(
                    dst_hbm.at[pl.ds(base0 + (k + 1) * CH, CH)],
                    didx.at[dk1], isem[dk1]).wait()
                issue_gathers(k + 1, bn, dk1)

            wait_gathers(k, b, dk)

            base = base0 + k * CH
            for g in range(CH // 16):
                a = asb[b, pl.ds(g * 16, 16)] + adb[b, pl.ds(g * 16, 16)]
                e = jnp.where(a >= 0, a, NEG_SLOPE * a)
                wv = jnp.exp(e)
                gid = base + g * 16 + lax.iota(jnp.int32, 16)
                wv = jnp.where(gid < E, wv, 0.0)
                wbuf[b, pl.ds(g * 16, 16)] = wv

            def rowgrp(g, carry):
                for i in range(16):
                    r = g * 16 + i
                    wsp = plsc.load_gather(wbuf.at[b],
                                           [jnp.full((16,), r, jnp.int32)])
                    for cc in range(D // 16):
                        rows[b, r, pl.ds(cc * 16, 16)] = (
                            rows[b, r, pl.ds(cc * 16, 16)] * wsp)
                return carry
            lax.fori_loop(0, CH // 16, rowgrp, 0)

            issue_scatters(b, dk)

        def chunk_body(k, carry):
            for u in range(NIB):
                @pl.when(lax.rem(k, NIB) == u)
                def _():
                    process(k, u)
            return carry

        lax.fori_loop(0, NCHUNK, chunk_body, 0)

        wait_scatters((NCHUNK - 2) % NBUF, (NCHUNK - 2) % NIB)
        wait_scatters((NCHUNK - 1) % NBUF, (NCHUNK - 1) % NIB)

        pltpu.sync_copy(rows.at[0, pl.ds(0, 16)], acc_sh.at[dumi], add=True)
        pltpu.sync_copy(wbuf.at[0, pl.ds(0, 16)], den_sh.at[dumi], add=True)
        plsc.subcore_barrier()

        pltpu.sync_copy(acc_sh.at[pl.ds(s * ROWS_PER_SUB, ROWS_PER_SUB)],
                        outp_hbm.at[c, pl.ds(s * ROWS_PER_SUB, ROWS_PER_SUB)])
        pltpu.sync_copy(den_sh.at[pl.ds(s * DEN_PER_SUB, DEN_PER_SUB)], den_v)
        pltpu.sync_copy(den_v,
                        denp_hbm.at[pl.ds(c * DEN_PAD + s * DEN_PER_SUB,
                                          DEN_PER_SUB)])

    return sc_edges


def kernel(x, edge_index, W, a_src, a_dst, bias):
    E = edge_index.shape[1]
    NCHUNK = (E + NW * CH - 1) // (NW * CH)
    EPAD = NW * CH * NCHUNK

    h, als, ald = _project(x, W, a_src, a_dst)

    pad = EPAD - E
    srcp = jnp.concatenate([edge_index[0], jnp.zeros((pad,), jnp.int32)])
    dstp = jnp.concatenate([edge_index[1], jnp.zeros((pad,), jnp.int32)])

    z2 = jnp.zeros((NPAD, D), jnp.float32)
    z1 = jnp.zeros((DEN_PAD,), jnp.float32)

    sc_edges = _make_sc_kernel(E, NCHUNK)
    parts, denp = sc_edges(als.reshape(N), ald.reshape(N), srcp, dstp, h,
                           z2, z1)

    d0 = denp[:N].reshape(N, 1)
    d1 = denp[DEN_PAD:DEN_PAD + N].reshape(N, 1)
    return _finalize(parts, d0, d1, als, ald, h, bias)
